# Initial kernel scaffold; baseline (speedup 1.0000x reference)
import jax
import jax.numpy as jnp
from jax.experimental import pallas as pl


def kernel(seq, seq_pos, emb_table, pos_table):
    raise NotImplementedError("write your pallas kernel here")


# speedup vs baseline: 2.5926x; 2.5926x over previous
import functools

import jax
import jax.numpy as jnp
from jax import lax
from jax.experimental import pallas as pl
from jax.experimental.pallas import tpu as pltpu
from jax.experimental.pallas import tpu_sc as plsc

B = 4096
L = 200
EMB = 32
N = B * L
NC = 2
NS = 16
NW = NC * NS
BPW = N // NW
CHUNK = 1024
G = CHUNK // 128
NCH = BPW // CHUNK
IDXROWS = BPW // 128


def _build():
    mesh = plsc.VectorSubcoreMesh(core_axis_name="c", subcore_axis_name="s")

    @functools.partial(
        pl.kernel,
        mesh=mesh,
        compiler_params=pltpu.CompilerParams(use_tc_tiling_on_sc=False),
        out_type=jax.ShapeDtypeStruct((N, EMB), jnp.float32),
        scratch_types=[
            pltpu.VMEM((G, 128), jnp.int32),
            pltpu.VMEM((G, 128), jnp.int32),
            pltpu.VMEM((CHUNK, EMB), jnp.float32),
            pltpu.VMEM((CHUNK, EMB), jnp.float32),
            pltpu.SemaphoreType.DMA,
        ],
    )
    def emb_add(seq_hbm, pidx_hbm, emb_hbm, ptab_hbm, out_hbm,
                sidx_v, pidx_v, erows_v, prows_v, sem):
        cid = lax.axis_index("c")
        sid = lax.axis_index("s")
        wid = sid * NC + cid
        row0 = wid * IDXROWS

        def chunk_body(k, _):
            r = row0 + k * G
            pltpu.sync_copy(seq_hbm.at[pl.ds(r, G)], sidx_v)
            pltpu.sync_copy(pidx_hbm.at[pl.ds(r, G)], pidx_v)
            handles = []
            for g in range(G):
                handles.append(pltpu.async_copy(
                    emb_hbm.at[sidx_v.at[g]],
                    erows_v.at[pl.ds(g * 128, 128)], sem))
                handles.append(pltpu.async_copy(
                    ptab_hbm.at[pidx_v.at[g]],
                    prows_v.at[pl.ds(g * 128, 128)], sem))
            for h in handles:
                h.wait()

            def add_body(j, _):
                a0 = erows_v[j, pl.ds(0, 16)]
                b0 = prows_v[j, pl.ds(0, 16)]
                erows_v[j, pl.ds(0, 16)] = a0 + b0
                a1 = erows_v[j, pl.ds(16, 16)]
                b1 = prows_v[j, pl.ds(16, 16)]
                erows_v[j, pl.ds(16, 16)] = a1 + b1
                return _

            lax.fori_loop(0, CHUNK, add_body, None)
            pltpu.sync_copy(
                erows_v, out_hbm.at[pl.ds(wid * BPW + k * CHUNK, CHUNK)])
            return _

        lax.fori_loop(0, NCH, chunk_body, None)

    return emb_add


_EMB_ADD = _build()


def kernel(seq, seq_pos, emb_table, pos_table):
    seq2 = seq.reshape(N // 128, 128).astype(jnp.int32)
    pos2 = seq_pos.reshape(N // 128, 128).astype(jnp.int32)
    out = _EMB_ADD(seq2, pos2, emb_table, pos_table)
    return out.reshape(B, L, EMB)


# speedup vs baseline: 2.6317x; 1.0151x over previous
import functools

import jax
import jax.numpy as jnp
from jax import lax
from jax.experimental import pallas as pl
from jax.experimental.pallas import tpu as pltpu
from jax.experimental.pallas import tpu_sc as plsc

B = 4096
L = 200
EMB = 32
N = B * L
NC = 2
NS = 16
NW = NC * NS
BPW = N // NW
CHUNK = 512
G = CHUNK // 128
NCH = BPW // CHUNK
IDXROWS = BPW // 128


def _build():
    mesh = plsc.VectorSubcoreMesh(core_axis_name="c", subcore_axis_name="s")

    @functools.partial(
        pl.kernel,
        mesh=mesh,
        compiler_params=pltpu.CompilerParams(use_tc_tiling_on_sc=False),
        out_type=jax.ShapeDtypeStruct((N, EMB), jnp.float32),
        scratch_types=[
            pltpu.VMEM((G, 128), jnp.int32),
            pltpu.VMEM((G, 128), jnp.int32),
            pltpu.VMEM((G, 128), jnp.int32),
            pltpu.VMEM((G, 128), jnp.int32),
            pltpu.VMEM((CHUNK, EMB), jnp.float32),
            pltpu.VMEM((CHUNK, EMB), jnp.float32),
            pltpu.VMEM((CHUNK, EMB), jnp.float32),
            pltpu.VMEM((CHUNK, EMB), jnp.float32),
            pltpu.VMEM((CHUNK, EMB), jnp.float32),
            pltpu.VMEM((CHUNK, EMB), jnp.float32),
            pltpu.SemaphoreType.DMA,
            pltpu.SemaphoreType.DMA,
            pltpu.SemaphoreType.DMA,
            pltpu.SemaphoreType.DMA,
        ],
    )
    def emb_add(seq_hbm, pidx_hbm, emb_hbm, ptab_hbm, out_hbm,
                sidx0, sidx1, qidx0, qidx1, e0, e1, p0, p1, o0, o1,
                gsem0, gsem1, osem0, osem1):
        sidx = (sidx0, sidx1)
        qidx = (qidx0, qidx1)
        ero = (e0, e1)
        pro = (p0, p1)
        obu = (o0, o1)
        gsem = (gsem0, gsem1)
        osem = (osem0, osem1)

        cid = lax.axis_index("c")
        sid = lax.axis_index("s")
        wid = sid * NC + cid
        row0 = wid * IDXROWS
        base = wid * BPW

        def stage(k, b):
            r = row0 + k * G
            pltpu.sync_copy(seq_hbm.at[pl.ds(r, G)], sidx[b])
            pltpu.sync_copy(pidx_hbm.at[pl.ds(r, G)], qidx[b])
            for g in range(G):
                pltpu.async_copy(emb_hbm.at[sidx[b].at[g]],
                                 ero[b].at[pl.ds(g * 128, 128)], gsem[b])
                pltpu.async_copy(ptab_hbm.at[qidx[b].at[g]],
                                 pro[b].at[pl.ds(g * 128, 128)], gsem[b])

        def drain_gathers(b):
            for g in range(G):
                pltpu.make_async_copy(
                    emb_hbm.at[sidx[b].at[g]],
                    ero[b].at[pl.ds(g * 128, 128)], gsem[b]).wait()
                pltpu.make_async_copy(
                    ptab_hbm.at[qidx[b].at[g]],
                    pro[b].at[pl.ds(g * 128, 128)], gsem[b]).wait()

        def drain_out(b):
            pltpu.make_async_copy(
                obu[b], out_hbm.at[pl.ds(base, CHUNK)], osem[b]).wait()

        def add_and_out(k, b):
            drain_gathers(b)

            @pl.when(k >= 2)
            def _():
                drain_out(b)

            @plsc.parallel_loop(0, CHUNK, unroll=8)
            def _(j):
                obu[b][j, pl.ds(0, 16)] = (
                    ero[b][j, pl.ds(0, 16)] + pro[b][j, pl.ds(0, 16)])
                obu[b][j, pl.ds(16, 16)] = (
                    ero[b][j, pl.ds(16, 16)] + pro[b][j, pl.ds(16, 16)])

            pltpu.async_copy(
                obu[b], out_hbm.at[pl.ds(base + k * CHUNK, CHUNK)], osem[b])

        stage(0, 0)

        def outer(k2, _):
            for b in range(2):
                k = k2 * 2 + b

                @pl.when(k + 1 < NCH)
                def _():
                    stage(k + 1, 1 - b)

                add_and_out(k, b)
            return _

        lax.fori_loop(0, NCH // 2, outer, None)
        drain_out(0)
        drain_out(1)

    return emb_add


_EMB_ADD = _build()


def kernel(seq, seq_pos, emb_table, pos_table):
    seq2 = seq.reshape(N // 128, 128).astype(jnp.int32)
    pos2 = seq_pos.reshape(N // 128, 128).astype(jnp.int32)
    out = _EMB_ADD(seq2, pos2, emb_table, pos_table)
    return out.reshape(B, L, EMB)


# speedup vs baseline: 2.6337x; 1.0008x over previous
import functools

import jax
import jax.numpy as jnp
from jax import lax
from jax.experimental import pallas as pl
from jax.experimental.pallas import tpu as pltpu
from jax.experimental.pallas import tpu_sc as plsc

B = 4096
L = 200
EMB = 32
N = B * L
NC = 2
NS = 16
NW = NC * NS
PER = N // NW
CH = 256
NCH = PER // CH


def _build():
    mesh = plsc.VectorSubcoreMesh(core_axis_name="c", subcore_axis_name="s")

    @functools.partial(
        pl.kernel,
        mesh=mesh,
        compiler_params=pltpu.CompilerParams(use_tc_tiling_on_sc=False),
        out_type=jax.ShapeDtypeStruct((N, EMB), jnp.float32),
        scratch_types=[
            pltpu.VMEM((PER,), jnp.int32),
            pltpu.VMEM((PER,), jnp.int32),
            pltpu.VMEM((CH, EMB), jnp.float32),
            pltpu.VMEM((CH, EMB), jnp.float32),
            pltpu.VMEM((CH, EMB), jnp.float32),
            pltpu.VMEM((CH, EMB), jnp.float32),
            pltpu.VMEM((CH, EMB), jnp.float32),
            pltpu.VMEM((CH, EMB), jnp.float32),
            pltpu.SemaphoreType.DMA,
            pltpu.SemaphoreType.DMA,
            pltpu.SemaphoreType.DMA,
            pltpu.SemaphoreType.DMA,
        ],
    )
    def emb_add(seq_hbm, pidx_hbm, emb_hbm, ptab_hbm, out_hbm,
                sidx, qidx, e0, e1, p0, p1, s0, s1,
                gsem0, gsem1, osem0, osem1):
        ebuf = (e0, e1)
        pbuf = (p0, p1)
        sbuf = (s0, s1)
        gsem = (gsem0, gsem1)
        osem = (osem0, osem1)

        cid = lax.axis_index("c")
        sid = lax.axis_index("s")
        base = (sid * NC + cid) * PER

        pltpu.sync_copy(seq_hbm.at[pl.ds(base, PER)], sidx)
        pltpu.sync_copy(pidx_hbm.at[pl.ds(base, PER)], qidx)

        def stage(c, b):
            off = c * CH
            pltpu.async_copy(emb_hbm.at[sidx.at[pl.ds(off, CH)]],
                             ebuf[b], gsem[b])
            pltpu.async_copy(ptab_hbm.at[qidx.at[pl.ds(off, CH)]],
                             pbuf[b], gsem[b])

        def wait_gathers(c, b):
            off = c * CH
            pltpu.make_async_copy(emb_hbm.at[sidx.at[pl.ds(off, CH)]],
                                  ebuf[b], gsem[b]).wait()
            pltpu.make_async_copy(ptab_hbm.at[qidx.at[pl.ds(off, CH)]],
                                  pbuf[b], gsem[b]).wait()

        def wait_out(c, b):
            pltpu.make_async_copy(sbuf[b],
                                  out_hbm.at[pl.ds(base + c * CH, CH)],
                                  osem[b]).wait()

        def process(c, b):
            wait_gathers(c, b)

            @pl.when(c >= 2)
            def _():
                wait_out(c - 2, b)

            @plsc.parallel_loop(0, CH, unroll=8)
            def _(j):
                sbuf[b][j, pl.ds(0, 16)] = (ebuf[b][j, pl.ds(0, 16)] +
                                            pbuf[b][j, pl.ds(0, 16)])
                sbuf[b][j, pl.ds(16, 16)] = (ebuf[b][j, pl.ds(16, 16)] +
                                             pbuf[b][j, pl.ds(16, 16)])

            pltpu.async_copy(sbuf[b],
                             out_hbm.at[pl.ds(base + c * CH, CH)],
                             osem[b])

        stage(0, 0)

        def outer(c2, carry):
            for b in range(2):
                c = c2 * 2 + b

                @pl.when(c + 1 < NCH)
                def _():
                    stage(c + 1, 1 - b)

                process(c, b)
            return carry

        lax.fori_loop(0, NCH // 2, outer, None)
        wait_out(NCH - 2, 0)
        wait_out(NCH - 1, 1)

    return emb_add


_EMB_ADD = _build()


def kernel(seq, seq_pos, emb_table, pos_table):
    seq_f = seq.reshape(N).astype(jnp.int32)
    pos_f = seq_pos.reshape(N).astype(jnp.int32)
    out = _EMB_ADD(seq_f, pos_f, emb_table, pos_table)
    return out.reshape(B, L, EMB)


# speedup vs baseline: 3.2937x; 1.2506x over previous
import functools

import jax
import jax.numpy as jnp
from jax import lax
from jax.experimental import pallas as pl
from jax.experimental.pallas import tpu as pltpu
from jax.experimental.pallas import tpu_sc as plsc

B = 4096
L = 200
EMB = 32
N = B * L
NC = 2
NS = 16
NW = NC * NS
BPW = B // NW
PER = BPW * L
CH = L
NCH = BPW
NSLOT = 4


def _build():
    mesh = plsc.VectorSubcoreMesh(core_axis_name="c", subcore_axis_name="s")

    @functools.partial(
        pl.kernel,
        mesh=mesh,
        compiler_params=pltpu.CompilerParams(use_tc_tiling_on_sc=False),
        out_type=jax.ShapeDtypeStruct((B, L, 128), jnp.float32),
        scratch_types=(
            [pltpu.VMEM((PER,), jnp.int32),
             pltpu.VMEM((PER,), jnp.int32)] +
            [pltpu.VMEM((CH, EMB), jnp.float32) for _ in range(NSLOT)] +
            [pltpu.SemaphoreType.DMA for _ in range(3 * NSLOT)]
        ),
    )
    def emb_add(seq_hbm, pidx_hbm, emb_hbm, ptab_hbm, out_hbm,
                sidx, qidx, b0, b1, b2, b3, *sems):
        buf = (b0, b1, b2, b3)
        psem = sems[0:NSLOT]
        asem = sems[NSLOT:2 * NSLOT]
        osem = sems[2 * NSLOT:3 * NSLOT]

        cid = lax.axis_index("c")
        sid = lax.axis_index("s")
        wid = sid * NC + cid
        base = wid * PER
        bbase = wid * BPW

        pltpu.sync_copy(seq_hbm.at[pl.ds(base, PER)], sidx)
        pltpu.sync_copy(pidx_hbm.at[pl.ds(base, PER)], qidx)

        def pos_gather(c, s):
            pltpu.async_copy(ptab_hbm.at[qidx.at[pl.ds(c * CH, CH)]],
                             buf[s], psem[s])

        def wait_pos(c, s):
            pltpu.make_async_copy(ptab_hbm.at[qidx.at[pl.ds(c * CH, CH)]],
                                  buf[s], psem[s]).wait()

        def emb_gather_add(c, s):
            pltpu.async_copy(emb_hbm.at[sidx.at[pl.ds(c * CH, CH)]],
                             buf[s], asem[s], add=True)

        def wait_emb(c, s):
            pltpu.make_async_copy(emb_hbm.at[sidx.at[pl.ds(c * CH, CH)]],
                                  buf[s], asem[s]).wait()

        def out_copy(c, s):
            pltpu.async_copy(buf[s],
                             out_hbm.at[bbase + c, :, pl.ds(0, EMB)],
                             osem[s])

        def wait_out(c, s):
            pltpu.make_async_copy(buf[s],
                                  out_hbm.at[bbase + c, :, pl.ds(0, EMB)],
                                  osem[s]).wait()

        pos_gather(0, 0)
        pos_gather(1, 1)

        def body(c, k):
            @pl.when(c >= NSLOT - 2)
            def _():
                wait_out(c - (NSLOT - 2), (k + 2) % NSLOT)

            @pl.when(c + 2 < NCH)
            def _():
                pos_gather(c + 2, (k + 2) % NSLOT)

            wait_pos(c, k)
            emb_gather_add(c, k)

            @pl.when(c >= 1)
            def _():
                wait_emb(c - 1, (k + NSLOT - 1) % NSLOT)
                out_copy(c - 1, (k + NSLOT - 1) % NSLOT)

        def outer(c4, carry):
            for k in range(NSLOT):
                body(c4 * NSLOT + k, k)
            return carry

        lax.fori_loop(0, NCH // NSLOT, outer, None)

        wait_emb(NCH - 1, (NCH - 1) % NSLOT)
        out_copy(NCH - 1, (NCH - 1) % NSLOT)
        wait_out(NCH - 2, (NCH - 2) % NSLOT)
        wait_out(NCH - 1, (NCH - 1) % NSLOT)

    return emb_add


_EMB_ADD = _build()


def kernel(seq, seq_pos, emb_table, pos_table):
    seq_f = seq.reshape(N).astype(jnp.int32)
    pos_f = seq_pos.reshape(N).astype(jnp.int32)
    out = _EMB_ADD(seq_f, pos_f, emb_table, pos_table)
    return out[:, :, :EMB]
